# baseline (speedup 1.0000x reference)
import functools

import jax
import jax.numpy as jnp
from jax import lax
from jax.experimental import pallas as pl
from jax.experimental.pallas import tpu as pltpu
from jax.experimental.pallas import tpu_sc as plsc

K = 1024
D = 64
BETA = 0.25
BN = 4096

_NC = 2
_NS = 16
_NW = _NC * _NS
_L = 16
_CHUNK = 128


def _vq_idx_block(z_ref, w_ref, z2_ref, w2_ref, idx_ref, acc_ref):
    z = z_ref[...]
    w = w_ref[...]
    s = jax.lax.dot_general(z, w, (((1,), (1,)), ((), ())),
                            preferred_element_type=jnp.float32)
    z2 = z2_ref[...]
    w2 = w2_ref[...]
    dist = (z2 - 2.0 * s) + w2
    d_min = jnp.min(dist, axis=1, keepdims=True)
    lane = jax.lax.broadcasted_iota(jnp.int32, (BN, K), 1)
    idx = jnp.min(jnp.where(dist == d_min, lane, K), axis=1)
    idx_ref[...] = idx

    @pl.when(pl.program_id(0) == 0)
    def _init():
        acc_ref[...] = jnp.zeros((1, 1), jnp.float32)

    acc_ref[...] += jnp.sum(d_min).reshape(1, 1)


def _sc_gather(w_flat, idx):
    n = idx.shape[0]
    per_w = n // _NW
    n_chunks = per_w // _CHUNK
    mesh = plsc.VectorSubcoreMesh(core_axis_name="c", subcore_axis_name="s")

    @functools.partial(
        pl.kernel, mesh=mesh,
        out_type=jax.ShapeDtypeStruct((n, D), jnp.float32),
        compiler_params=pltpu.CompilerParams(needs_layout_passes=False),
        scratch_types=[
            pltpu.VMEM((K * D,), jnp.float32),
            pltpu.VMEM((per_w,), jnp.int32),
            pltpu.VMEM((_CHUNK, D), jnp.float32),
        ],
    )
    def k(w_hbm, idx_hbm, out_hbm, w_v, idx_v, buf):
        wid = lax.axis_index("s") * _NC + lax.axis_index("c")
        tbase = wid * per_w
        pltpu.sync_copy(w_hbm, w_v)
        pltpu.sync_copy(idx_hbm.at[pl.ds(tbase, per_w)], idx_v)
        lanes = lax.iota(jnp.int32, _L)

        def chunk_body(chunk, _):
            for g in range(_CHUNK // _L):
                idxg = idx_v[pl.ds(chunk * _CHUNK + g * _L, _L)] * D
                rows = lanes + g * _L
                for c in range(D):
                    val = plsc.load_gather(w_v, [idxg + c])
                    plsc.store_scatter(
                        buf, [rows, jnp.full((_L,), c, jnp.int32)], val)
            pltpu.sync_copy(
                buf, out_hbm.at[pl.ds(tbase + chunk * _CHUNK, _CHUNK)])
            return _

        lax.fori_loop(0, n_chunks, chunk_body, 0)

    return k(w_flat, idx)


@jax.jit
def kernel(z_e, W):
    n, d = z_e.shape
    grid = n // BN
    z2 = (z_e ** 2).sum(-1, keepdims=True)
    w2 = (W ** 2).sum(-1)[None, :]
    idx, acc = pl.pallas_call(
        _vq_idx_block,
        grid=(grid,),
        in_specs=[
            pl.BlockSpec((BN, d), lambda i: (i, 0)),
            pl.BlockSpec((K, d), lambda i: (0, 0)),
            pl.BlockSpec((BN, 1), lambda i: (i, 0)),
            pl.BlockSpec((1, K), lambda i: (0, 0)),
        ],
        out_specs=[
            pl.BlockSpec((BN,), lambda i: (i,)),
            pl.BlockSpec((1, 1), lambda i: (0, 0)),
        ],
        out_shape=[
            jax.ShapeDtypeStruct((n,), jnp.int32),
            jax.ShapeDtypeStruct((1, 1), jnp.float32),
        ],
    )(z_e, W, z2, w2)
    commitment = BETA * acc[0, 0] / (n * d)
    zq = _sc_gather(W.reshape(-1), idx)
    return (zq, commitment, idx)


# speedup vs baseline: 1.3800x; 1.3800x over previous
import functools

import jax
import jax.numpy as jnp
from jax import lax
from jax.experimental import pallas as pl
from jax.experimental.pallas import tpu as pltpu
from jax.experimental.pallas import tpu_sc as plsc

K = 1024
D = 64
BETA = 0.25
BN = 4096

_NC = 2
_NS = 16
_NW = _NC * _NS
_L = 16
_CHUNK = 128


def _vq_idx_block(z_ref, w_ref, z2_ref, w2_ref, idx_ref, acc_ref):
    z = z_ref[...]
    w = w_ref[...]
    s = jax.lax.dot_general(z, w, (((1,), (1,)), ((), ())),
                            preferred_element_type=jnp.float32)
    z2 = z2_ref[...]
    w2 = w2_ref[...]
    dist = (z2 - 2.0 * s) + w2
    d_min = jnp.min(dist, axis=1, keepdims=True)
    lane = jax.lax.broadcasted_iota(jnp.int32, (BN, K), 1)
    idx = jnp.min(jnp.where(dist == d_min, lane, K), axis=1)
    idx_ref[...] = idx

    @pl.when(pl.program_id(0) == 0)
    def _init():
        acc_ref[...] = jnp.zeros((1, 1), jnp.float32)

    acc_ref[...] += jnp.sum(d_min).reshape(1, 1)


def _sc_gather(w_flat, idx):
    n = idx.shape[0]
    per_w = n // _NW
    n_chunks = per_w // _CHUNK
    mesh = plsc.VectorSubcoreMesh(core_axis_name="c", subcore_axis_name="s")

    @functools.partial(
        pl.kernel, mesh=mesh,
        out_type=jax.ShapeDtypeStruct((n, D), jnp.float32),
        compiler_params=pltpu.CompilerParams(needs_layout_passes=False),
        scratch_types=[
            pltpu.VMEM((K * D,), jnp.float32),
            pltpu.VMEM((per_w,), jnp.int32),
            pltpu.VMEM((_CHUNK, D), jnp.float32),
        ],
    )
    def k(w_hbm, idx_hbm, out_hbm, w_v, idx_v, buf):
        wid = lax.axis_index("s") * _NC + lax.axis_index("c")
        tbase = wid * per_w
        pltpu.sync_copy(w_hbm, w_v)
        pltpu.sync_copy(idx_hbm.at[pl.ds(tbase, per_w)], idx_v)
        lanes = lax.iota(jnp.int32, _L)

        def chunk_body(chunk, _):
            for g in range(_CHUNK // _L):
                idxg = idx_v[pl.ds(chunk * _CHUNK + g * _L, _L)] * D
                for t in range(_L):
                    base = lax.gather(
                        idxg, jnp.full((_L, 1), t, jnp.int32),
                        lax.GatherDimensionNumbers(
                            offset_dims=(), collapsed_slice_dims=(0,),
                            start_index_map=(0,)),
                        (1,), mode=lax.GatherScatterMode.PROMISE_IN_BOUNDS)
                    for c in range(D // _L):
                        val = plsc.load_gather(
                            w_v, [base + (lanes + c * _L)])
                        buf[g * _L + t, pl.ds(c * _L, _L)] = val
            pltpu.sync_copy(
                buf, out_hbm.at[pl.ds(tbase + chunk * _CHUNK, _CHUNK)])
            return _

        lax.fori_loop(0, n_chunks, chunk_body, 0)

    return k(w_flat, idx)


@jax.jit
def kernel(z_e, W):
    n, d = z_e.shape
    grid = n // BN
    z2 = (z_e ** 2).sum(-1, keepdims=True)
    w2 = (W ** 2).sum(-1)[None, :]
    idx, acc = pl.pallas_call(
        _vq_idx_block,
        grid=(grid,),
        in_specs=[
            pl.BlockSpec((BN, d), lambda i: (i, 0)),
            pl.BlockSpec((K, d), lambda i: (0, 0)),
            pl.BlockSpec((BN, 1), lambda i: (i, 0)),
            pl.BlockSpec((1, K), lambda i: (0, 0)),
        ],
        out_specs=[
            pl.BlockSpec((BN,), lambda i: (i,)),
            pl.BlockSpec((1, 1), lambda i: (0, 0)),
        ],
        out_shape=[
            jax.ShapeDtypeStruct((n,), jnp.int32),
            jax.ShapeDtypeStruct((1, 1), jnp.float32),
        ],
    )(z_e, W, z2, w2)
    commitment = BETA * acc[0, 0] / (n * d)
    zq = _sc_gather(W.reshape(-1), idx)
    return (zq, commitment, idx)


# speedup vs baseline: 1.3872x; 1.0053x over previous
import functools

import jax
import jax.numpy as jnp
from jax import lax
from jax.experimental import pallas as pl
from jax.experimental.pallas import tpu as pltpu
from jax.experimental.pallas import tpu_sc as plsc

K = 1024
D = 64
BETA = 0.25
BN = 4096

_NC = 2
_NS = 16
_NW = _NC * _NS
_L = 16
_CHUNK = 128


def _vq_idx_block(z_ref, w_ref, z2_ref, w2_ref, idx_ref, acc_ref):
    z = z_ref[...]
    w = w_ref[...]
    s = jax.lax.dot_general(z, w, (((1,), (1,)), ((), ())),
                            preferred_element_type=jnp.float32)
    z2 = z2_ref[...]
    w2 = w2_ref[...]
    dist = (z2 - 2.0 * s) + w2
    d_min = jnp.min(dist, axis=1, keepdims=True)
    lane = jax.lax.broadcasted_iota(jnp.int32, (BN, K), 1)
    idx = jnp.min(jnp.where(dist == d_min, lane, K), axis=1)
    idx_ref[...] = idx

    @pl.when(pl.program_id(0) == 0)
    def _init():
        acc_ref[...] = jnp.zeros((1, 1), jnp.float32)

    acc_ref[...] += jnp.sum(d_min).reshape(1, 1)


def _sc_gather(w_flat, idx):
    n = idx.shape[0]
    per_w = n // _NW
    n_chunks = per_w // _CHUNK
    mesh = plsc.VectorSubcoreMesh(core_axis_name="c", subcore_axis_name="s")

    @functools.partial(
        pl.kernel, mesh=mesh,
        out_type=jax.ShapeDtypeStruct((n, D), jnp.float32),
        compiler_params=pltpu.CompilerParams(needs_layout_passes=False),
        scratch_types=[
            pltpu.VMEM((K * D,), jnp.float32),
            pltpu.VMEM((per_w,), jnp.int32),
            pltpu.VMEM((_CHUNK, D), jnp.float32),
            pltpu.VMEM((_CHUNK, D), jnp.float32),
            pltpu.SemaphoreType.DMA,
            pltpu.SemaphoreType.DMA,
        ],
    )
    def k(w_hbm, idx_hbm, out_hbm, w_v, idx_v, buf0, buf1, sem0, sem1):
        wid = lax.axis_index("s") * _NC + lax.axis_index("c")
        tbase = wid * per_w
        pltpu.sync_copy(w_hbm, w_v)
        pltpu.sync_copy(idx_hbm.at[pl.ds(tbase, per_w)], idx_v)
        lanes = lax.iota(jnp.int32, _L)

        def fill(buf, chunk):
            for g in range(_CHUNK // _L):
                idxg = idx_v[pl.ds(chunk * _CHUNK + g * _L, _L)] * D
                for t in range(_L):
                    base = lax.gather(
                        idxg, jnp.full((_L, 1), t, jnp.int32),
                        lax.GatherDimensionNumbers(
                            offset_dims=(), collapsed_slice_dims=(0,),
                            start_index_map=(0,)),
                        (1,), mode=lax.GatherScatterMode.PROMISE_IN_BOUNDS)
                    for c in range(D // _L):
                        val = plsc.load_gather(
                            w_v, [base + (lanes + c * _L)])
                        buf[g * _L + t, pl.ds(c * _L, _L)] = val

        def out_at(chunk):
            return out_hbm.at[pl.ds(tbase + chunk * _CHUNK, _CHUNK)]

        def pair_body(j, _):
            @pl.when(j > 0)
            def _w0():
                pltpu.make_async_copy(buf0, out_at(2 * j), sem0).wait()

            fill(buf0, 2 * j)
            pltpu.async_copy(buf0, out_at(2 * j), sem0)

            @pl.when(j > 0)
            def _w1():
                pltpu.make_async_copy(buf1, out_at(2 * j + 1), sem1).wait()

            fill(buf1, 2 * j + 1)
            pltpu.async_copy(buf1, out_at(2 * j + 1), sem1)
            return _

        lax.fori_loop(0, n_chunks // 2, pair_body, 0)
        pltpu.make_async_copy(buf0, out_at(n_chunks - 2), sem0).wait()
        pltpu.make_async_copy(buf1, out_at(n_chunks - 1), sem1).wait()

    return k(w_flat, idx)


@jax.jit
def kernel(z_e, W):
    n, d = z_e.shape
    grid = n // BN
    z2 = (z_e ** 2).sum(-1, keepdims=True)
    w2 = (W ** 2).sum(-1)[None, :]
    idx, acc = pl.pallas_call(
        _vq_idx_block,
        grid=(grid,),
        in_specs=[
            pl.BlockSpec((BN, d), lambda i: (i, 0)),
            pl.BlockSpec((K, d), lambda i: (0, 0)),
            pl.BlockSpec((BN, 1), lambda i: (i, 0)),
            pl.BlockSpec((1, K), lambda i: (0, 0)),
        ],
        out_specs=[
            pl.BlockSpec((BN,), lambda i: (i,)),
            pl.BlockSpec((1, 1), lambda i: (0, 0)),
        ],
        out_shape=[
            jax.ShapeDtypeStruct((n,), jnp.int32),
            jax.ShapeDtypeStruct((1, 1), jnp.float32),
        ],
    )(z_e, W, z2, w2)
    commitment = BETA * acc[0, 0] / (n * d)
    zq = _sc_gather(W.reshape(-1), idx)
    return (zq, commitment, idx)


# speedup vs baseline: 1.4988x; 1.0804x over previous
import functools

import jax
import jax.numpy as jnp
from jax import lax
from jax.experimental import pallas as pl
from jax.experimental.pallas import tpu as pltpu
from jax.experimental.pallas import tpu_sc as plsc

K = 1024
D = 64
BETA = 0.25
BN = 4096

_NC = 2
_NS = 16
_NW = _NC * _NS
_L = 16
_CHUNK = 128


def _vq_idx_block(z_ref, w_ref, z2_ref, w2_ref, idx_ref, acc_ref):
    z = z_ref[...]
    w = w_ref[...]
    s = jax.lax.dot_general(z, w, (((1,), (1,)), ((), ())),
                            preferred_element_type=jnp.float32)
    z2 = z2_ref[...]
    w2 = w2_ref[...]
    dist = (z2 - 2.0 * s) + w2
    d_min = jnp.min(dist, axis=1, keepdims=True)
    lane = jax.lax.broadcasted_iota(jnp.int32, (BN, K), 1)
    idx = jnp.min(jnp.where(dist == d_min, lane, K), axis=1)
    idx_ref[...] = idx

    @pl.when(pl.program_id(0) == 0)
    def _init():
        acc_ref[...] = jnp.zeros((1, 1), jnp.float32)

    acc_ref[...] += jnp.sum(d_min).reshape(1, 1)


def _sc_gather(w_flat, idx):
    n = idx.shape[0]
    per_w = n // _NW
    n_chunks = per_w // _CHUNK
    mesh = plsc.VectorSubcoreMesh(core_axis_name="c", subcore_axis_name="s")

    @functools.partial(
        pl.kernel, mesh=mesh,
        out_type=jax.ShapeDtypeStruct((n, D), jnp.float32),
        compiler_params=pltpu.CompilerParams(needs_layout_passes=False),
        scratch_types=[
            pltpu.VMEM((K * D,), jnp.float32),
            pltpu.VMEM((per_w,), jnp.int32),
            pltpu.VMEM((_CHUNK, D), jnp.float32),
            pltpu.VMEM((_CHUNK, D), jnp.float32),
            pltpu.SemaphoreType.DMA,
            pltpu.SemaphoreType.DMA,
        ],
    )
    def k(w_hbm, idx_hbm, out_hbm, w_v, idx_v, buf0, buf1, sem0, sem1):
        wid = lax.axis_index("s") * _NC + lax.axis_index("c")
        tbase = wid * per_w
        pltpu.sync_copy(w_hbm, w_v)
        pltpu.sync_copy(idx_hbm.at[pl.ds(tbase, per_w)], idx_v)
        lanes = lax.iota(jnp.int32, _L)

        def fill(buf, chunk):
            @plsc.parallel_loop(0, _CHUNK // _L, 1, unroll=2)
            def _group(g):
                idxg = idx_v[pl.ds(chunk * _CHUNK + g * _L, _L)] * D
                for t in range(_L):
                    base = lax.gather(
                        idxg, jnp.full((_L, 1), t, jnp.int32),
                        lax.GatherDimensionNumbers(
                            offset_dims=(), collapsed_slice_dims=(0,),
                            start_index_map=(0,)),
                        (1,), mode=lax.GatherScatterMode.PROMISE_IN_BOUNDS)
                    for c in range(D // _L):
                        val = plsc.load_gather(
                            w_v, [base + (lanes + c * _L)])
                        buf[g * _L + t, pl.ds(c * _L, _L)] = val

        def out_at(chunk):
            return out_hbm.at[pl.ds(tbase + chunk * _CHUNK, _CHUNK)]

        def pair_body(j, _):
            @pl.when(j > 0)
            def _w0():
                pltpu.make_async_copy(buf0, out_at(2 * j), sem0).wait()

            fill(buf0, 2 * j)
            pltpu.async_copy(buf0, out_at(2 * j), sem0)

            @pl.when(j > 0)
            def _w1():
                pltpu.make_async_copy(buf1, out_at(2 * j + 1), sem1).wait()

            fill(buf1, 2 * j + 1)
            pltpu.async_copy(buf1, out_at(2 * j + 1), sem1)
            return _

        lax.fori_loop(0, n_chunks // 2, pair_body, 0)
        pltpu.make_async_copy(buf0, out_at(n_chunks - 2), sem0).wait()
        pltpu.make_async_copy(buf1, out_at(n_chunks - 1), sem1).wait()

    return k(w_flat, idx)


@jax.jit
def kernel(z_e, W):
    n, d = z_e.shape
    grid = n // BN
    z2 = (z_e ** 2).sum(-1, keepdims=True)
    w2 = (W ** 2).sum(-1)[None, :]
    idx, acc = pl.pallas_call(
        _vq_idx_block,
        grid=(grid,),
        in_specs=[
            pl.BlockSpec((BN, d), lambda i: (i, 0)),
            pl.BlockSpec((K, d), lambda i: (0, 0)),
            pl.BlockSpec((BN, 1), lambda i: (i, 0)),
            pl.BlockSpec((1, K), lambda i: (0, 0)),
        ],
        out_specs=[
            pl.BlockSpec((BN,), lambda i: (i,)),
            pl.BlockSpec((1, 1), lambda i: (0, 0)),
        ],
        out_shape=[
            jax.ShapeDtypeStruct((n,), jnp.int32),
            jax.ShapeDtypeStruct((1, 1), jnp.float32),
        ],
    )(z_e, W, z2, w2)
    commitment = BETA * acc[0, 0] / (n * d)
    zq = _sc_gather(W.reshape(-1), idx)
    return (zq, commitment, idx)


# speedup vs baseline: 1.6062x; 1.0716x over previous
import functools

import jax
import jax.numpy as jnp
from jax import lax
from jax.experimental import pallas as pl
from jax.experimental.pallas import tpu as pltpu
from jax.experimental.pallas import tpu_sc as plsc

K = 1024
D = 64
BETA = 0.25
BN = 4096

_NC = 2
_NS = 16
_NW = _NC * _NS
_L = 16
_CHUNK = 128


def _vq_idx_block(z_ref, w_ref, z2_ref, w2_ref, idx_ref, acc_ref):
    z = z_ref[...]
    w = w_ref[...]
    s = jax.lax.dot_general(z, w, (((1,), (1,)), ((), ())),
                            preferred_element_type=jnp.float32)
    z2 = z2_ref[...]
    w2 = w2_ref[...]
    dist = (z2 - 2.0 * s) + w2
    d_min = jnp.min(dist, axis=1, keepdims=True)
    lane = jax.lax.broadcasted_iota(jnp.int32, (BN, K), 1)
    idx = jnp.min(jnp.where(dist == d_min, lane, K), axis=1)
    idx_ref[...] = idx

    @pl.when(pl.program_id(0) == 0)
    def _init():
        acc_ref[...] = jnp.zeros((1, 1), jnp.float32)

    acc_ref[...] += jnp.sum(d_min).reshape(1, 1)


def _sc_gather(w_flat, idx):
    n = idx.shape[0]
    per_w = n // _NW
    n_chunks = per_w // _CHUNK
    mesh = plsc.VectorSubcoreMesh(core_axis_name="c", subcore_axis_name="s")

    @functools.partial(
        pl.kernel, mesh=mesh,
        out_type=jax.ShapeDtypeStruct((n, D), jnp.float32),
        compiler_params=pltpu.CompilerParams(needs_layout_passes=False),
        scratch_types=[
            pltpu.VMEM((K * D,), jnp.float32),
            pltpu.VMEM((per_w,), jnp.int32),
            pltpu.VMEM((_CHUNK, D), jnp.float32),
            pltpu.VMEM((_CHUNK, D), jnp.float32),
            pltpu.SemaphoreType.DMA,
            pltpu.SemaphoreType.DMA,
        ],
    )
    def k(w_hbm, idx_hbm, out_hbm, w_v, idx_v, buf0, buf1, sem0, sem1):
        wid = lax.axis_index("s") * _NC + lax.axis_index("c")
        tbase = wid * per_w
        pltpu.sync_copy(w_hbm, w_v)
        pltpu.sync_copy(idx_hbm.at[pl.ds(tbase, per_w)], idx_v)
        lanes = lax.iota(jnp.int32, _L)

        def fill(buf, chunk):
            @plsc.parallel_loop(0, _CHUNK // _L, 1, unroll=4)
            def _group(g):
                idxg = idx_v[pl.ds(chunk * _CHUNK + g * _L, _L)] * D
                for t in range(_L):
                    base = lax.gather(
                        idxg, jnp.full((_L, 1), t, jnp.int32),
                        lax.GatherDimensionNumbers(
                            offset_dims=(), collapsed_slice_dims=(0,),
                            start_index_map=(0,)),
                        (1,), mode=lax.GatherScatterMode.PROMISE_IN_BOUNDS)
                    for c in range(D // _L):
                        val = plsc.load_gather(
                            w_v, [base + (lanes + c * _L)])
                        buf[g * _L + t, pl.ds(c * _L, _L)] = val

        def out_at(chunk):
            return out_hbm.at[pl.ds(tbase + chunk * _CHUNK, _CHUNK)]

        def pair_body(j, _):
            @pl.when(j > 0)
            def _w0():
                pltpu.make_async_copy(buf0, out_at(2 * j), sem0).wait()

            fill(buf0, 2 * j)
            pltpu.async_copy(buf0, out_at(2 * j), sem0)

            @pl.when(j > 0)
            def _w1():
                pltpu.make_async_copy(buf1, out_at(2 * j + 1), sem1).wait()

            fill(buf1, 2 * j + 1)
            pltpu.async_copy(buf1, out_at(2 * j + 1), sem1)
            return _

        lax.fori_loop(0, n_chunks // 2, pair_body, 0)
        pltpu.make_async_copy(buf0, out_at(n_chunks - 2), sem0).wait()
        pltpu.make_async_copy(buf1, out_at(n_chunks - 1), sem1).wait()

    return k(w_flat, idx)


@jax.jit
def kernel(z_e, W):
    n, d = z_e.shape
    grid = n // BN
    z2 = (z_e ** 2).sum(-1, keepdims=True)
    w2 = (W ** 2).sum(-1)[None, :]
    idx, acc = pl.pallas_call(
        _vq_idx_block,
        grid=(grid,),
        in_specs=[
            pl.BlockSpec((BN, d), lambda i: (i, 0)),
            pl.BlockSpec((K, d), lambda i: (0, 0)),
            pl.BlockSpec((BN, 1), lambda i: (i, 0)),
            pl.BlockSpec((1, K), lambda i: (0, 0)),
        ],
        out_specs=[
            pl.BlockSpec((BN,), lambda i: (i,)),
            pl.BlockSpec((1, 1), lambda i: (0, 0)),
        ],
        out_shape=[
            jax.ShapeDtypeStruct((n,), jnp.int32),
            jax.ShapeDtypeStruct((1, 1), jnp.float32),
        ],
    )(z_e, W, z2, w2)
    commitment = BETA * acc[0, 0] / (n * d)
    zq = _sc_gather(W.reshape(-1), idx)
    return (zq, commitment, idx)
